# Initial kernel scaffold; baseline (speedup 1.0000x reference)
import jax
import jax.numpy as jnp
from jax.experimental import pallas as pl


def kernel(x, embed, Wr_s, Wr_v, Wh, Wv, Wsv, Wro_s, Wro_v):
    raise NotImplementedError("write your pallas kernel here")


# speedup vs baseline: 1631.4748x; 1631.4748x over previous
import jax
import jax.numpy as jnp
from jax.experimental import pallas as pl
from jax.experimental.pallas import tpu as pltpu

N = 512
T = 2
B = 10
F = 64
V = 16
FI = 32
RV = 8
R_MAX = 5.0
CUT = 1000000.0
AVG = 511.0


def _mace_kernel(x_ref, xT_ref, embed_ref, Wr_s_ref, Wr_v_ref, Wh_ref,
                 Wv_ref, Wsv_ref, Wro_s_ref, Wro_v_ref,
                 vec_out_ref, inv_out_ref,
                 rbcat, qc, hb):
    f32 = jnp.float32
    x = x_ref[:]

    d2 = jnp.zeros((N, N), dtype=f32)
    for c in range(3):
        col = x_ref[:, c:c + 1]
        row = xT_ref[c:c + 1, :]
        diff = row - col
        d2 = d2 + diff * diff
    r = jnp.sqrt(d2 + 1e-9)
    invr = 1.0 / r

    u_env = 2.0 * (1.0 - r * (1.0 / CUT))
    safe = jnp.where(u_env > 0, u_env, 1.0)
    env = jnp.where(u_env > 0, 1.2 * jnp.exp(-1.0 / safe), 0.0)

    rows = jax.lax.broadcasted_iota(jnp.int32, (N, N), 0)
    cols = jax.lax.broadcasted_iota(jnp.int32, (N, N), 1)
    diag = rows == cols
    base = jnp.where(diag, 0.0, jnp.sqrt(2.0 / R_MAX) * env * invr)

    theta = (jnp.pi / R_MAX) * r
    s_cur = jnp.sin(theta)
    c1x2 = 2.0 * jnp.cos(theta)
    s_prev = jnp.zeros((N, N), dtype=f32)

    for b in range(B):
        plane = base * s_cur
        rbcat[:, b * N:(b + 1) * N] = plane
        pp = plane * invr
        Y = jnp.dot(pp, x, preferred_element_type=f32)
        S = jnp.sum(pp, axis=1, keepdims=True)
        for c in range(3):
            qc[c, :, b:b + 1] = x_ref[:, c:c + 1] * S - Y[:, c:c + 1]
        s_cur, s_prev = c1x2 * s_cur - s_prev, s_cur

    inv_avg = 1.0 / AVG
    h = jnp.broadcast_to(embed_ref[0:1, :], (N, F))
    vf = [jnp.zeros((N, V), dtype=f32) for _ in range(3)]
    for t in range(T):
        for b in range(B):
            hb[b * N:(b + 1) * N, :] = h * Wr_s_ref[t, b:b + 1, :]
        agg_s = jnp.dot(rbcat[:], hb[:], preferred_element_type=f32) * inv_avg
        Wr_v_t = Wr_v_ref[t]
        Wv_t = Wv_ref[t]
        for c in range(3):
            agg_v = jnp.dot(qc[c], Wr_v_t, preferred_element_type=f32) * inv_avg
            vf[c] = vf[c] + jnp.dot(agg_v, Wv_t, preferred_element_type=f32)
        vnorm = vf[0] * vf[0] + vf[1] * vf[1] + vf[2] * vf[2]
        h = jnp.tanh(jnp.dot(agg_s, Wh_ref[t], preferred_element_type=f32)
                     + jnp.dot(vnorm, Wsv_ref[t], preferred_element_type=f32)) + h

    inv_out_ref[:] = jnp.dot(h, Wro_s_ref[:], preferred_element_type=f32)
    for c in range(3):
        com_c = jnp.sum(x_ref[:, c:c + 1]) * (1.0 / N)
        vec_out_ref[c] = jnp.dot(vf[c], Wro_v_ref[:],
                                 preferred_element_type=f32) + com_c


def kernel(x, embed, Wr_s, Wr_v, Wh, Wv, Wsv, Wro_s, Wro_v):
    f32 = jnp.float32
    xT = x.T
    vec3, inv = pl.pallas_call(
        _mace_kernel,
        out_shape=(
            jax.ShapeDtypeStruct((3, N, RV), f32),
            jax.ShapeDtypeStruct((N, FI), f32),
        ),
        in_specs=[pl.BlockSpec(memory_space=pltpu.VMEM) for _ in range(10)],
        out_specs=(pl.BlockSpec(memory_space=pltpu.VMEM),
                   pl.BlockSpec(memory_space=pltpu.VMEM)),
        scratch_shapes=[
            pltpu.VMEM((N, B * N), f32),
            pltpu.VMEM((3, N, B), f32),
            pltpu.VMEM((B * N, F), f32),
        ],
    )(x, xT, embed, Wr_s, Wr_v, Wh, Wv, Wsv, Wro_s, Wro_v)
    return jnp.transpose(vec3, (1, 2, 0)), inv


# speedup vs baseline: 1783.1219x; 1.0930x over previous
import jax
import jax.numpy as jnp
from jax.experimental import pallas as pl
from jax.experimental.pallas import tpu as pltpu

N = 512
T = 2
B = 10
F = 64
V = 16
FI = 32
RV = 8
R_MAX = 5.0
CUT = 1000000.0
AVG = 511.0

_TWO_OPI = 0.6366197723675814
_PIO2_HI = 1.57079637050628662109375
_PIO2_LO = -4.37113900018624283e-8


def _sincos(theta):
    q = jnp.round(theta * _TWO_OPI)
    qi = q.astype(jnp.int32)
    y = (theta - q * _PIO2_HI) - q * _PIO2_LO
    y2 = y * y
    ps = -1.9840874e-4 + y2 * 2.7525562e-6
    ps = 8.3333310e-3 + y2 * ps
    ps = -0.16666667 + y2 * ps
    sp = y + y * (y2 * ps)
    pc = 2.439044879e-5 * y2 - 1.388731625e-3
    pc = 4.16666418e-2 + y2 * pc
    pc = -0.5 + y2 * pc
    cp = 1.0 + y2 * pc
    swap = (qi & 1) == 1
    s_neg = (qi & 2) != 0
    c_neg = ((qi + 1) & 2) != 0
    s = jnp.where(swap, cp, sp)
    c = jnp.where(swap, sp, cp)
    s = jnp.where(s_neg, -s, s)
    c = jnp.where(c_neg, -c, c)
    return s, c


def _mace_kernel(x_ref, xT_ref, embed_ref, Wr_s_ref, Wr_v_ref, Wh_ref,
                 Wv_ref, Wsv_ref, Wro_s_ref, Wro_v_ref,
                 vec_out_ref, inv_out_ref,
                 rbcat, qc, hb, x4):
    f32 = jnp.float32
    x = x_ref[:]
    x4[:, 0:3] = x
    x4[:, 3:4] = jnp.ones((N, 1), dtype=f32)

    rows = jax.lax.broadcasted_iota(jnp.int32, (N, N), 0)
    cols = jax.lax.broadcasted_iota(jnp.int32, (N, N), 1)
    diag = rows == cols
    d2 = jnp.zeros((N, N), dtype=f32)
    for c in range(3):
        diff = xT_ref[c:c + 1, :] - x_ref[:, c:c + 1]
        d2 = d2 + diff * diff
    r = jnp.sqrt(d2 + 1e-9)
    invr = 1.0 / r

    env = 1.2 * jnp.exp(-1.0 / (2.0 * (1.0 - r * (1.0 / CUT))))
    base = jnp.where(diag, 0.0, jnp.sqrt(2.0 / R_MAX) * env * invr)

    s_cur, c1 = _sincos((jnp.pi / R_MAX) * r)
    c1x2 = 2.0 * c1
    s_prev = jnp.zeros((N, N), dtype=f32)

    for b in range(B):
        plane = base * s_cur
        rbcat[:, b * N:(b + 1) * N] = plane
        pp = plane * invr
        Y = jnp.dot(pp, x4[:], preferred_element_type=f32)
        S = Y[:, 3:4]
        for c in range(3):
            qc[c, :, b:b + 1] = x_ref[:, c:c + 1] * S - Y[:, c:c + 1]
        s_cur, s_prev = c1x2 * s_cur - s_prev, s_cur

    inv_avg = 1.0 / AVG
    h = jnp.broadcast_to(embed_ref[0:1, :], (N, F))
    vf = [jnp.zeros((N, V), dtype=f32) for _ in range(3)]
    for t in range(T):
        for b in range(B):
            hb[b * N:(b + 1) * N, :] = h * Wr_s_ref[t, b:b + 1, :]
        agg_s = jnp.dot(rbcat[:], hb[:], preferred_element_type=f32) * inv_avg
        Wr_v_t = Wr_v_ref[t]
        Wv_t = Wv_ref[t]
        for c in range(3):
            agg_v = jnp.dot(qc[c], Wr_v_t, preferred_element_type=f32) * inv_avg
            vf[c] = vf[c] + jnp.dot(agg_v, Wv_t, preferred_element_type=f32)
        vnorm = vf[0] * vf[0] + vf[1] * vf[1] + vf[2] * vf[2]
        h = jnp.tanh(jnp.dot(agg_s, Wh_ref[t], preferred_element_type=f32)
                     + jnp.dot(vnorm, Wsv_ref[t], preferred_element_type=f32)) + h

    inv_out_ref[:] = jnp.dot(h, Wro_s_ref[:], preferred_element_type=f32)
    for c in range(3):
        com_c = jnp.sum(x_ref[:, c:c + 1]) * (1.0 / N)
        vo = jnp.dot(vf[c], Wro_v_ref[:], preferred_element_type=f32) + com_c
        for rv in range(RV):
            vec_out_ref[:, rv * 3 + c:rv * 3 + c + 1] = vo[:, rv:rv + 1]


def kernel(x, embed, Wr_s, Wr_v, Wh, Wv, Wsv, Wro_s, Wro_v):
    f32 = jnp.float32
    vec24, inv = pl.pallas_call(
        _mace_kernel,
        out_shape=(
            jax.ShapeDtypeStruct((N, RV * 3), f32),
            jax.ShapeDtypeStruct((N, FI), f32),
        ),
        in_specs=[pl.BlockSpec(memory_space=pltpu.VMEM) for _ in range(10)],
        out_specs=(pl.BlockSpec(memory_space=pltpu.VMEM),
                   pl.BlockSpec(memory_space=pltpu.VMEM)),
        scratch_shapes=[
            pltpu.VMEM((N, B * N), f32),
            pltpu.VMEM((3, N, B), f32),
            pltpu.VMEM((B * N, F), f32),
            pltpu.VMEM((N, 4), f32),
        ],
    )(x, x.T, embed, Wr_s, Wr_v, Wh, Wv, Wsv, Wro_s, Wro_v)
    return vec24.reshape(N, RV, 3), inv


# speedup vs baseline: 1854.3133x; 1.0399x over previous
import jax
import jax.numpy as jnp
from jax.experimental import pallas as pl
from jax.experimental.pallas import tpu as pltpu

N = 512
T = 2
B = 10
F = 64
V = 16
FI = 32
RV = 8
R_MAX = 5.0
CUT = 1000000.0
AVG = 511.0

_TWO_OPI = 0.6366197723675814
_PIO2_HI = 1.57079637050628662109375
_PIO2_LO = -4.37113900018624283e-8


def _sincos(theta):
    q = jnp.round(theta * _TWO_OPI)
    qi = q.astype(jnp.int32)
    y = (theta - q * _PIO2_HI) - q * _PIO2_LO
    y2 = y * y
    ps = -1.9840874e-4 + y2 * 2.7525562e-6
    ps = 8.3333310e-3 + y2 * ps
    ps = -0.16666667 + y2 * ps
    sp = y + y * (y2 * ps)
    pc = 2.439044879e-5 * y2 - 1.388731625e-3
    pc = 4.16666418e-2 + y2 * pc
    pc = -0.5 + y2 * pc
    cp = 1.0 + y2 * pc
    swap = (qi & 1) == 1
    s_neg = (qi & 2) != 0
    c_neg = ((qi + 1) & 2) != 0
    s = jnp.where(swap, cp, sp)
    c = jnp.where(swap, sp, cp)
    s = jnp.where(s_neg, -s, s)
    c = jnp.where(c_neg, -c, c)
    return s, c


def _mace_kernel(x_ref, xT_ref, embed_ref, Wr_s_ref, Wr_v_ref, Wh_ref,
                 Wv_ref, Wsv_ref, Wro_s_ref, Wro_v_ref,
                 vec_out_ref, inv_out_ref,
                 rbcat, qc, hb, x4, w3):
    f32 = jnp.float32
    bf16 = jnp.bfloat16
    x = x_ref[:]
    xT = xT_ref[:]
    x4[:, 0:3] = x.astype(bf16)
    x4[:, 3:4] = jnp.ones((N, 1), dtype=bf16)

    gram = jnp.dot(x, xT, preferred_element_type=f32)
    sq_col = jnp.sum(x * x, axis=1, keepdims=True)
    sq_row = jnp.sum(xT * xT, axis=0, keepdims=True)
    d2 = jnp.maximum(sq_col + sq_row - 2.0 * gram, 0.0)
    r = jnp.sqrt(d2 + 1e-9)
    invr = 1.0 / r

    rows = jax.lax.broadcasted_iota(jnp.int32, (N, N), 0)
    cols = jax.lax.broadcasted_iota(jnp.int32, (N, N), 1)
    diag = rows == cols

    env = 1.2 * jnp.exp(-1.0 / (2.0 * (1.0 - r * (1.0 / CUT))))
    base = jnp.where(diag, 0.0, jnp.sqrt(2.0 / R_MAX) * env * invr)

    s_cur, c1 = _sincos((jnp.pi / R_MAX) * r)
    c1x2 = 2.0 * c1
    s_prev = jnp.zeros((N, N), dtype=f32)

    for b in range(B):
        plane = base * s_cur
        rbcat[:, b * N:(b + 1) * N] = plane.astype(bf16)
        pp = (plane * invr).astype(bf16)
        Y = jnp.dot(pp, x4[:], preferred_element_type=f32)
        S = Y[:, 3:4]
        for c in range(3):
            qc[c, :, b:b + 1] = x_ref[:, c:c + 1] * S - Y[:, c:c + 1]
        s_cur, s_prev = c1x2 * s_cur - s_prev, s_cur

    inv_avg = 1.0 / AVG
    h = jnp.broadcast_to(embed_ref[0:1, :], (N, F))
    vf = [jnp.zeros((N, V), dtype=f32) for _ in range(3)]
    for t in range(T):
        for b in range(B):
            hb[b * N:(b + 1) * N, :] = (h * Wr_s_ref[t, b:b + 1, :]).astype(bf16)
        agg_s = jnp.dot(rbcat[:], hb[:], preferred_element_type=f32) * inv_avg
        Wr_v_t = Wr_v_ref[t]
        Wv_t = Wv_ref[t]
        for c in range(3):
            agg_v = jnp.dot(qc[c], Wr_v_t, preferred_element_type=f32) * inv_avg
            vf[c] = vf[c] + jnp.dot(agg_v, Wv_t, preferred_element_type=f32)
        vnorm = vf[0] * vf[0] + vf[1] * vf[1] + vf[2] * vf[2]
        h = jnp.tanh(jnp.dot(agg_s, Wh_ref[t], preferred_element_type=f32)
                     + jnp.dot(vnorm, Wsv_ref[t], preferred_element_type=f32)) + h

    inv_out_ref[:] = jnp.dot(h, Wro_s_ref[:], preferred_element_type=f32)
    w3[:] = jnp.zeros((3 * V, RV * 3), dtype=f32)
    wro = Wro_v_ref[:]
    for c in range(3):
        for rv in range(RV):
            w3[c * V:(c + 1) * V, rv * 3 + c:rv * 3 + c + 1] = wro[:, rv:rv + 1]
    vfcat = jnp.concatenate(vf, axis=1)
    lane = jax.lax.broadcasted_iota(jnp.int32, (1, RV * 3), 1)
    modpat = lane - 3 * jnp.floor(lane.astype(f32) * (1.0 / 3.0)).astype(jnp.int32)
    com = [jnp.sum(xT[c:c + 1, :]) * (1.0 / N) for c in range(3)]
    comvec = jnp.where(modpat == 0, com[0],
                       jnp.where(modpat == 1, com[1], com[2]))
    vec_out_ref[:] = (jnp.dot(vfcat, w3[:], preferred_element_type=f32)
                      + comvec)


def kernel(x, embed, Wr_s, Wr_v, Wh, Wv, Wsv, Wro_s, Wro_v):
    f32 = jnp.float32
    vec24, inv = pl.pallas_call(
        _mace_kernel,
        out_shape=(
            jax.ShapeDtypeStruct((N, RV * 3), f32),
            jax.ShapeDtypeStruct((N, FI), f32),
        ),
        in_specs=[pl.BlockSpec(memory_space=pltpu.VMEM) for _ in range(10)],
        out_specs=(pl.BlockSpec(memory_space=pltpu.VMEM),
                   pl.BlockSpec(memory_space=pltpu.VMEM)),
        scratch_shapes=[
            pltpu.VMEM((N, B * N), jnp.bfloat16),
            pltpu.VMEM((3, N, B), f32),
            pltpu.VMEM((B * N, F), jnp.bfloat16),
            pltpu.VMEM((N, 4), jnp.bfloat16),
            pltpu.VMEM((3 * V, RV * 3), f32),
        ],
    )(x, x.T, embed, Wr_s, Wr_v, Wh, Wv, Wsv, Wro_s, Wro_v)
    return vec24.reshape(N, RV, 3), inv


# speedup vs baseline: 1883.5293x; 1.0158x over previous
import jax
import jax.numpy as jnp
from jax.experimental import pallas as pl
from jax.experimental.pallas import tpu as pltpu

N = 512
T = 2
B = 10
F = 64
V = 16
FI = 32
RV = 8
R_MAX = 5.0
CUT = 1000000.0
AVG = 511.0

_TWO_OPI = 0.6366197723675814
_PIO2_HI = 1.57079637050628662109375
_PIO2_LO = -4.37113900018624283e-8


def _sincos(theta):
    q = jnp.round(theta * _TWO_OPI)
    qi = q.astype(jnp.int32)
    y = (theta - q * _PIO2_HI) - q * _PIO2_LO
    y2 = y * y
    ps = -1.9840874e-4 + y2 * 2.7525562e-6
    ps = 8.3333310e-3 + y2 * ps
    ps = -0.16666667 + y2 * ps
    sp = y + y * (y2 * ps)
    pc = 2.439044879e-5 * y2 - 1.388731625e-3
    pc = 4.16666418e-2 + y2 * pc
    pc = -0.5 + y2 * pc
    cp = 1.0 + y2 * pc
    swap = (qi & 1) == 1
    s_neg = (qi & 2) != 0
    c_neg = ((qi + 1) & 2) != 0
    s = jnp.where(swap, cp, sp)
    c = jnp.where(swap, sp, cp)
    s = jnp.where(s_neg, -s, s)
    c = jnp.where(c_neg, -c, c)
    return s, c


def _mace_kernel(x_ref, xT_ref, embed_ref, Wr_s_ref, Wr_v_ref, Wh_ref,
                 Wv_ref, Wsv_ref, Wro_s_ref, Wro_v_ref,
                 vec_out_ref, inv_out_ref,
                 rbcat, qc, hb, x4, w3):
    f32 = jnp.float32
    bf16 = jnp.bfloat16
    x = x_ref[:]
    xT = xT_ref[:]
    x4[:, 0:3] = x
    x4[:, 3:4] = jnp.ones((N, 1), dtype=f32)

    rows = jax.lax.broadcasted_iota(jnp.int32, (N, N), 0)
    cols = jax.lax.broadcasted_iota(jnp.int32, (N, N), 1)
    diag = rows == cols
    d2 = jnp.zeros((N, N), dtype=f32)
    for c in range(3):
        diff = xT_ref[c:c + 1, :] - x_ref[:, c:c + 1]
        d2 = d2 + diff * diff
    r = jnp.sqrt(d2 + 1e-9)
    invr = 1.0 / r

    env = 1.2 * jnp.exp(-1.0 / (2.0 * (1.0 - r * (1.0 / CUT))))
    base = jnp.where(diag, 0.0, jnp.sqrt(2.0 / R_MAX) * env * invr)

    s_cur, c1 = _sincos((jnp.pi / R_MAX) * r)
    c1x2 = 2.0 * c1
    s_prev = jnp.zeros((N, N), dtype=f32)

    for b in range(B):
        plane = base * s_cur
        rbcat[:, b * N:(b + 1) * N] = plane.astype(bf16)
        pp = plane * invr
        Y = jnp.dot(pp, x4[:], preferred_element_type=f32)
        S = Y[:, 3:4]
        for c in range(3):
            qc[c, :, b:b + 1] = x_ref[:, c:c + 1] * S - Y[:, c:c + 1]
        s_cur, s_prev = c1x2 * s_cur - s_prev, s_cur

    inv_avg = 1.0 / AVG
    h = jnp.broadcast_to(embed_ref[0:1, :], (N, F))
    vf = [jnp.zeros((N, V), dtype=f32) for _ in range(3)]
    for t in range(T):
        for b in range(B):
            hb[b * N:(b + 1) * N, :] = (h * Wr_s_ref[t, b:b + 1, :]).astype(bf16)
        agg_s = jnp.dot(rbcat[:], hb[:], preferred_element_type=f32) * inv_avg
        Wr_v_t = Wr_v_ref[t]
        Wv_t = Wv_ref[t]
        for c in range(3):
            agg_v = jnp.dot(qc[c], Wr_v_t, preferred_element_type=f32) * inv_avg
            vf[c] = vf[c] + jnp.dot(agg_v, Wv_t, preferred_element_type=f32)
        vnorm = vf[0] * vf[0] + vf[1] * vf[1] + vf[2] * vf[2]
        h = jnp.tanh(jnp.dot(agg_s, Wh_ref[t], preferred_element_type=f32)
                     + jnp.dot(vnorm, Wsv_ref[t], preferred_element_type=f32)) + h

    inv_out_ref[:] = jnp.dot(h, Wro_s_ref[:], preferred_element_type=f32)
    w3[:] = jnp.zeros((3 * V, RV * 3), dtype=f32)
    wro = Wro_v_ref[:]
    for c in range(3):
        for rv in range(RV):
            w3[c * V:(c + 1) * V, rv * 3 + c:rv * 3 + c + 1] = wro[:, rv:rv + 1]
    vfcat = jnp.concatenate(vf, axis=1)
    lane = jax.lax.broadcasted_iota(jnp.int32, (1, RV * 3), 1)
    modpat = lane - 3 * jnp.floor(lane.astype(f32) * (1.0 / 3.0)).astype(jnp.int32)
    com = [jnp.sum(xT[c:c + 1, :]) * (1.0 / N) for c in range(3)]
    comvec = jnp.where(modpat == 0, com[0],
                       jnp.where(modpat == 1, com[1], com[2]))
    vec_out_ref[:] = (jnp.dot(vfcat, w3[:], preferred_element_type=f32)
                      + comvec)


def kernel(x, embed, Wr_s, Wr_v, Wh, Wv, Wsv, Wro_s, Wro_v):
    f32 = jnp.float32
    vec24, inv = pl.pallas_call(
        _mace_kernel,
        out_shape=(
            jax.ShapeDtypeStruct((N, RV * 3), f32),
            jax.ShapeDtypeStruct((N, FI), f32),
        ),
        in_specs=[pl.BlockSpec(memory_space=pltpu.VMEM) for _ in range(10)],
        out_specs=(pl.BlockSpec(memory_space=pltpu.VMEM),
                   pl.BlockSpec(memory_space=pltpu.VMEM)),
        scratch_shapes=[
            pltpu.VMEM((N, B * N), jnp.bfloat16),
            pltpu.VMEM((3, N, B), f32),
            pltpu.VMEM((B * N, F), jnp.bfloat16),
            pltpu.VMEM((N, 4), f32),
            pltpu.VMEM((3 * V, RV * 3), f32),
        ],
    )(x, x.T, embed, Wr_s, Wr_v, Wh, Wv, Wsv, Wro_s, Wro_v)
    return vec24.reshape(N, RV, 3), inv
